# baseline (speedup 1.0000x reference)
import functools

import jax
import jax.numpy as jnp
from jax import lax
from jax.experimental import pallas as pl
from jax.experimental.pallas import tpu as pltpu
from jax.experimental.pallas import tpu_sc as plsc

_NC = 2
_NS = 16
_NW = _NC * _NS

_K = 128


def _sc_gather_pack(table2, idx1, extra2, ch, n):
    mesh = plsc.VectorSubcoreMesh(core_axis_name="c", subcore_axis_name="s")
    er = _K // 8

    @functools.partial(
        pl.kernel,
        out_type=jax.ShapeDtypeStruct((n, 128), jnp.float32),
        mesh=mesh,
        scratch_types=[
            pltpu.VMEM((2, _K), jnp.int32),
            pltpu.VMEM((2, _K), jnp.int32),
            pltpu.VMEM((2, _K, 128), jnp.float32),
            pltpu.VMEM((2, er, 128), jnp.float32),
            pltpu.VMEM((2, _K, 128), jnp.float32),
        ] + [pltpu.SemaphoreType.DMA] * 6,
    )
    def k(table_hbm, idx_hbm, extra_hbm, out_hbm,
          ivec, gv, slab, exv, outv, *sems):
        sem_idx = sems[0:2]
        sem_gat = sems[2:4]
        sem_st = sems[4:6]
        wid = lax.axis_index("s") * _NC + lax.axis_index("c")
        base = wid * (ch * _K)
        ebase = wid * (ch * er)

        @pl.loop(0, _K)
        def _(l):
            for bb in range(2):
                for c3 in range(3):
                    outv[bb, l, pl.ds(80 + 16 * c3, 16)] = (
                        jnp.zeros((16,), jnp.float32))

        def stage_idx(j, b):
            pltpu.async_copy(idx_hbm.at[pl.ds(base + j * _K, _K)],
                             ivec.at[b], sem_idx[b])

        def fire_gather(j, b):
            @pl.loop(0, _K // 16)
            def _(q):
                v = ivec[b, pl.ds(q * 16, 16)]
                gv[b, pl.ds(q * 16, 16)] = lax.shift_right_logical(v, 1)
            pltpu.async_copy(table_hbm.at[gv.at[b]], slab.at[b], sem_gat[b])
            pltpu.async_copy(extra_hbm.at[pl.ds(ebase + j * er, er)],
                             exv.at[b], sem_gat[b])

        def process(j, b):
            pltpu.make_async_copy(
                table_hbm.at[gv.at[b]], slab.at[b], sem_gat[b]).wait()
            pltpu.make_async_copy(
                extra_hbm.at[pl.ds(0, er)], exv.at[b], sem_gat[b]).wait()

            @pl.loop(0, _K // 16)
            def _(q):
                v = ivec[b, pl.ds(q * 16, 16)]
                for l in range(16):
                    s = v[l]
                    off = lax.mul(lax.bitwise_and(s, 1), 64)
                    tok = q * 16 + l
                    for c in range(4):
                        outv[b, tok, pl.ds(16 * c, 16)] = (
                            slab[b, tok, pl.ds(off + 16 * c, 16)])
                    outv[b, tok, pl.ds(64, 16)] = (
                        exv[b, lax.div(tok, 8),
                            pl.ds(lax.mul(lax.rem(tok, 8), 16), 16)])
            pltpu.async_copy(outv.at[b],
                             out_hbm.at[pl.ds(base + j * _K, _K)], sem_st[b])

        stage_idx(0, 0)
        pltpu.make_async_copy(
            idx_hbm.at[pl.ds(0, _K)], ivec.at[0], sem_idx[0]).wait()
        fire_gather(0, 0)
        stage_idx(1, 1)

        @pl.loop(0, ch, step=2)
        def _(g):
            for b in range(2):
                j = g + b
                o = 1 - b
                @pl.when(j + 1 < ch)
                def _():
                    pltpu.make_async_copy(
                        idx_hbm.at[pl.ds(0, _K)], ivec.at[o], sem_idx[o]).wait()
                    fire_gather(j + 1, o)
                @pl.when(j >= 2)
                def _():
                    pltpu.make_async_copy(
                        outv.at[b], out_hbm.at[pl.ds(0, _K)], sem_st[b]).wait()
                process(j, b)
                @pl.when(j + 2 < ch)
                def _():
                    stage_idx(j + 2, b)

        for b in range(2):
            @pl.when(ch - 2 + b >= 0)
            def _():
                pltpu.make_async_copy(
                    outv.at[b], out_hbm.at[pl.ds(0, _K)], sem_st[b]).wait()

    return k(table2, idx1, extra2)


def _tc_mlp(h, wfull, bb, n):
    tb = 2048

    def body(h_ref, w_ref, b_ref, o_ref):
        o_ref[...] = jnp.dot(h_ref[...], w_ref[...],
                             preferred_element_type=jnp.float32) + b_ref[...]

    return pl.pallas_call(
        body,
        grid=(n // tb,),
        in_specs=[
            pl.BlockSpec((tb, 128), lambda i: (i, 0)),
            pl.BlockSpec((128, 128), lambda i: (0, 0)),
            pl.BlockSpec((1, 128), lambda i: (0, 0)),
        ],
        out_specs=pl.BlockSpec((tb, 128), lambda i: (i, 0)),
        out_shape=jax.ShapeDtypeStruct((n, 128), jnp.float32),
        compiler_params=pltpu.CompilerParams(
            dimension_semantics=("arbitrary",),
        ),
    )(h, wfull, bb)


def kernel(x, extra_features, table, W, b):
    idx = x.reshape(-1).astype(jnp.int32)
    n = idx.shape[0]
    d = table.shape[1]
    e = extra_features.shape[1]
    ch = n // (_NW * _K)

    table2 = table.reshape(table.shape[0] // 2, 2 * d)
    extra2 = extra_features.reshape(n * e // 128, 128)

    h = _sc_gather_pack(table2, idx, extra2, ch, n)

    wfull = jnp.zeros((128, 128), jnp.float32).at[:d + e, :d].set(W)
    bb = jnp.zeros((1, 128), jnp.float32).at[0, :d].set(b)

    out128 = _tc_mlp(h, wfull, bb, n)
    return out128[:, :d]


# speedup vs baseline: 1.7865x; 1.7865x over previous
import functools

import jax
import jax.numpy as jnp
from jax import lax
from jax.experimental import pallas as pl
from jax.experimental.pallas import tpu as pltpu
from jax.experimental.pallas import tpu_sc as plsc

_NC = 2
_NS = 16
_NW = _NC * _NS

_K = 128


def _sc_gather(table, idx1, ch, n, d):
    mesh = plsc.VectorSubcoreMesh(core_axis_name="c", subcore_axis_name="s")

    @functools.partial(
        pl.kernel,
        out_type=jax.ShapeDtypeStruct((n, d), jnp.float32),
        mesh=mesh,
        scratch_types=[
            pltpu.VMEM((2, _K), jnp.int32),
            pltpu.VMEM((2, _K, 64), jnp.float32),
        ] + [pltpu.SemaphoreType.DMA] * 6,
    )
    def k(table_hbm, idx_hbm, out_hbm, ivec, rows, *sems):
        sem_idx = sems[0:2]
        sem_gat = sems[2:4]
        sem_st = sems[4:6]
        wid = lax.axis_index("s") * _NC + lax.axis_index("c")
        base = wid * (ch * _K)

        def stage_idx(j, b):
            pltpu.async_copy(idx_hbm.at[pl.ds(base + j * _K, _K)],
                             ivec.at[b], sem_idx[b])

        def fire_gather(b):
            @pl.loop(0, _K // 16)
            def _(q):
                v = ivec[b, pl.ds(q * 16, 16)]
                for l in range(16):
                    pltpu.async_copy(
                        table_hbm.at[v[l]], rows.at[b, q * 16 + l],
                        sem_gat[b],
                    )

        def wait_gather(b):
            pltpu.make_async_copy(
                table_hbm.at[pl.ds(0, _K)], rows.at[b], sem_gat[b]).wait()

        def store(j, b):
            pltpu.async_copy(rows.at[b],
                             out_hbm.at[pl.ds(base + j * _K, _K)], sem_st[b])

        def wait_store(b):
            pltpu.make_async_copy(
                rows.at[b], out_hbm.at[pl.ds(0, _K)], sem_st[b]).wait()

        stage_idx(0, 0)
        pltpu.make_async_copy(
            idx_hbm.at[pl.ds(0, _K)], ivec.at[0], sem_idx[0]).wait()
        fire_gather(0)
        stage_idx(1, 1)

        @pl.loop(0, ch, step=2)
        def _(g):
            for b in range(2):
                j = g + b
                o = 1 - b
                @pl.when(j + 1 < ch)
                def _():
                    pltpu.make_async_copy(
                        idx_hbm.at[pl.ds(0, _K)], ivec.at[o],
                        sem_idx[o]).wait()
                    @pl.when(j + 1 >= 2)
                    def _():
                        wait_store(o)
                    fire_gather(o)
                @pl.when(j + 2 < ch)
                def _():
                    stage_idx(j + 2, b)
                wait_gather(b)
                store(j, b)

        for b in range(2):
            wait_store(b)

    return k(table, idx1)


def _tc_mlp(h, extraT, wf, weT, bb, n):
    tb = 2048

    def body(h_ref, x_ref, w_ref, we_ref, b_ref, o_ref):
        acc = lax.dot_general(w_ref[...], h_ref[...],
                              (((0,), (1,)), ((), ())),
                              preferred_element_type=jnp.float32)
        acc += lax.dot_general(we_ref[...], x_ref[...],
                               (((0,), (0,)), ((), ())),
                               preferred_element_type=jnp.float32)
        o_ref[...] = acc + b_ref[...]

    return pl.pallas_call(
        body,
        grid=(n // tb,),
        in_specs=[
            pl.BlockSpec((tb, 64), lambda i: (i, 0)),
            pl.BlockSpec((16, tb), lambda i: (0, i)),
            pl.BlockSpec((64, 64), lambda i: (0, 0)),
            pl.BlockSpec((16, 64), lambda i: (0, 0)),
            pl.BlockSpec((64, 1), lambda i: (0, 0)),
        ],
        out_specs=pl.BlockSpec((64, tb), lambda i: (0, i)),
        out_shape=jax.ShapeDtypeStruct((64, n), jnp.float32),
        compiler_params=pltpu.CompilerParams(
            dimension_semantics=("arbitrary",),
        ),
    )(h, extraT, wf, weT, bb)


def kernel(x, extra_features, table, W, b):
    idx = x.reshape(-1).astype(jnp.int32)
    n = idx.shape[0]
    d = table.shape[1]
    ch = n // (_NW * _K)

    h = _sc_gather(table, idx, ch, n, d)
    extraT = extra_features.T

    wf = W[:d]
    weT = W[d:]
    bb = b.reshape(d, 1)

    outT = _tc_mlp(h, extraT, wf, weT, bb, n)
    return outT.T
